# baseline (speedup 1.0000x reference)
import functools

import jax
import jax.numpy as jnp
import numpy as np
from jax.experimental import pallas as pl
from jax.experimental.pallas import tpu as pltpu
from jax.experimental.pallas import tpu_sc as plsc

_N = 10000; _K = 32; _D = 128; _P = 64; _A = 4; _H = 8; _DH = 16
_NCH = 512; _NBA = 128; _DEPTH = 2
_NK = _N * _K
_BN = 200
_BE = _BN * _K
_GRID = _N // _BN
_CH = 80

_SEQ = pltpu.CompilerParams(dimension_semantics=("arbitrary",))


def _ln(x, g, b):
    m = jnp.mean(x, axis=-1, keepdims=True)
    v = jnp.mean((x - m) * (x - m), axis=-1, keepdims=True)
    return (x - m) * jax.lax.rsqrt(v + 1e-5) * g + b


def _expand(a, bn=_BN):
    return jnp.broadcast_to(a[:, None, :], (bn, _K, a.shape[-1])).reshape(bn * _K, a.shape[-1])


def _mm(a, b):
    return jax.lax.dot_general(
        a.astype(jnp.bfloat16), b.astype(jnp.bfloat16),
        (((1,), (0,)), ((), ())), preferred_element_type=jnp.float32)


_WIN = 8


def _sc_gather(table, idx, width, dtype):
    nk = idx.shape[0]
    info = plsc.get_sparse_core_info()
    nc, ns = info.num_cores, info.num_subcores
    nw = nc * ns
    per = nk // nw
    steps = per // _CH
    mesh = plsc.VectorSubcoreMesh(core_axis_name="c", subcore_axis_name="s")

    @functools.partial(
        pl.kernel, mesh=mesh,
        compiler_params=pltpu.CompilerParams(use_tc_tiling_on_sc=False),
        out_type=jax.ShapeDtypeStruct((nk, width), dtype),
        scratch_types=[
            pltpu.VMEM((per,), jnp.int32),
            pltpu.VMEM((2, _CH, width), dtype),
            pltpu.SemaphoreType.DMA,
            pltpu.SemaphoreType.DMA,
            pltpu.SemaphoreType.DMA,
            pltpu.SemaphoreType.DMA,
        ],
    )
    def gather_k(tab_h, idx_h, out_h, idx_v, rows_v, g0, g1, w0, w1):
        wid = jax.lax.axis_index("s") * nc + jax.lax.axis_index("c")
        base = wid * per
        pltpu.sync_copy(idx_h.at[pl.ds(base, per)], idx_v)
        gsem = (g0, g1)
        wsem = (w0, w1)

        def gath(c, b):
            return pltpu.make_async_copy(
                tab_h.at[idx_v.at[pl.ds(c * _CH, _CH)]],
                rows_v.at[b], gsem[b])

        def wb(c, b):
            return pltpu.make_async_copy(
                rows_v.at[b], out_h.at[pl.ds(base + c * _CH, _CH)], wsem[b])

        def pair(g, carry):
            c0 = 2 * g
            for b in (0, 1):
                @pl.when(g > 0)
                def _(b=b):
                    wb(0, b).wait()
                gath(c0 + b, b).start()
            for b in (0, 1):
                gath(0, b).wait()
                wb(c0 + b, b).start()
            return carry

        jax.lax.fori_loop(0, steps // 2, pair, 0)
        if steps % 2:
            c = steps - 1
            wb(0, 0).wait()
            gath(c, 0).start()
            gath(0, 0).wait()
            wb(c, 0).start()
        for b in (0, 1):
            wb(0, b).wait()

    return gather_k(table, idx)


def _pair_body(geo_e, geo_d,
               wrel, wdist, wdir, gp, bp, wp1, bp1, wp2, bp2, wb,
               scat, g3, r16, r3, sm8, s_re, s_ch, s_ba, cc, i66,
               bias_o, pm_o):
    ge = geo_e[...]
    gd = geo_d[...]
    delta = ge - _expand(gd)
    rel24 = jnp.clip(delta, -32.0, 32.0) + 32.0
    same24 = (delta == 0.0).astype(jnp.float32)
    relc = rel24 @ s_re[...]
    same = (same24 @ s_ch[...]) * (same24 @ s_ba[...])
    rel = jnp.where(same > 0.5, relc, 65.0)
    oh = (rel == i66[...]).astype(jnp.float32)
    pair = _mm(oh, wrel[...])

    pos_e = ge[:, 0:12]
    ca12 = _expand(gd @ scat[...])
    diff = pos_e - ca12
    d2 = (diff * diff) @ g3[...]
    dist = jnp.sqrt(d2)
    dist16 = dist @ r16[...]
    s = 22.0 / 16.0
    rbf = jnp.exp(-((dist16 - cc[...]) ** 2) * (1.0 / (2.0 * s * s)))
    pair = pair + _mm(rbf, wdist[...])
    dirs = diff * ((1.0 / (dist + 1e-6)) @ r3[...])
    pair = pair + _mm(dirs, wdir[...])

    pair = _ln(pair, gp[...], bp[...])
    pair = _mm(jax.nn.gelu(_mm(pair, wp1[...]) + bp1[...]), wp2[...]) + bp2[...]
    bias_o[...] = pair @ wb[...]
    pm_o[...] = (ge @ sm8[...]) * _expand(gd @ sm8[...])


def _qkv_body(x, gln1, bln1, wqkv, q_o, kv_o):
    h = _ln(x[...], gln1[...], bln1[...])
    qkv = _mm(h, wqkv[...])
    q_o[...] = qkv[:, 0:128]
    kv_o[...] = qkv[:, 128:384].astype(jnp.bfloat16)


def _attn_upd_body(q, kvn, bias_e, pm_e, x, mf2, ch2, ba2,
                   wo, gh, ght, gln2, bln2, w4,
                   x_o, out4_o, sb_o, sc_o):
    qe = _expand(q[...])
    ke = kvn[:, 0:128].astype(jnp.float32)
    ve = kvn[:, 128:256].astype(jnp.float32)
    lg8 = ((qe * ke) @ gh[...]) * (1.0 / np.sqrt(_DH)) + bias_e[...]
    pm8 = pm_e[...]
    lg8 = jnp.where(pm8 > 0.0, lg8, -1e9)
    l3 = lg8.reshape(_BN, _K, _H)
    m3 = jnp.max(l3, axis=1, keepdims=True)
    e3 = jnp.exp(l3 - m3)
    s3 = jnp.sum(e3, axis=1, keepdims=True)
    att = (e3 / s3).reshape(_BE, _H) * pm8
    w = (att @ ght[...]) * ve
    o = w.reshape(_BN, _K, _D).sum(axis=1)
    mf = mf2[...]
    xn = x[...] + (o @ wo[...]) * mf
    x_o[...] = xn

    i = pl.program_id(0)
    h = _ln(xn, gln2[...], bln2[...])
    big = h @ w4[...]
    u = big[:, 0:256]
    gat = jax.nn.gelu(big[:, 256:1024])
    out4_o[...] = jnp.concatenate([u, gat], axis=1)
    u2 = jnp.concatenate([u * mf, jnp.broadcast_to(mf, (_BN, 8))], axis=1)
    ohb = (ba2[...] == jax.lax.broadcasted_iota(jnp.int32, (1, _NBA), 1)
           ).astype(jnp.float32)
    ohc = (ch2[...] == jax.lax.broadcasted_iota(jnp.int32, (1, _NCH), 1)
           ).astype(jnp.float32)
    pb = jax.lax.dot_general(ohb, u2, (((0,), (0,)), ((), ())), preferred_element_type=jnp.float32)
    pc = jax.lax.dot_general(ohc, u2, (((0,), (0,)), ((), ())), preferred_element_type=jnp.float32)

    @pl.when(i == 0)
    def _():
        sb_o[...] = jnp.zeros_like(sb_o)
        sc_o[...] = jnp.zeros_like(sc_o)

    sb_o[...] += pb
    sc_o[...] += pc


def _mix_core(x, out4, ch2, ba2, mf2, sb, sc, wo2):
    o4 = out4[...]
    u = o4[:, 0:256]
    lg = o4[:, 256:512]
    cg = o4[:, 512:768]
    bg = o4[:, 768:1024]
    sbv = sb[...]
    scv = sc[...]
    meanb = sbv[:, 0:256] / jnp.maximum(sbv[:, 256:257], 1.0)
    meanc = scv[:, 0:256] / jnp.maximum(scv[:, 256:257], 1.0)
    ohb = (ba2[...] == jax.lax.broadcasted_iota(jnp.int32, (1, _NBA), 1)
           ).astype(jnp.float32)
    ohc = (ch2[...] == jax.lax.broadcasted_iota(jnp.int32, (1, _NCH), 1)
           ).astype(jnp.float32)
    hidden = bg * (ohb @ meanb) + cg * (ohc @ meanc) + lg * u
    return x[...] + (hidden @ wo2[...]) * mf2[...]


def _mix_qkv_body(x, out4, ch2, ba2, mf2, sb, sc, wo2, gln1, bln1, wqkv,
                  x_o, q_o, kv_o):
    xo = _mix_core(x, out4, ch2, ba2, mf2, sb, sc, wo2)
    x_o[...] = xo
    h = _ln(xo, gln1[...], bln1[...])
    qkv = _mm(h, wqkv[...])
    q_o[...] = qkv[:, 0:128]
    kv_o[...] = qkv[:, 128:384].astype(jnp.bfloat16)


def _mix_final_body(x, out4, ch2, ba2, mf2, sb, sc, wo2, glnf, blnf, x_o):
    xo = _mix_core(x, out4, ch2, ba2, mf2, sb, sc, wo2)
    x_o[...] = _ln(xo, glnf[...], blnf[...])


def _full(a):
    return pl.BlockSpec(a.shape, lambda i: (0,) * a.ndim)


def _nodes(width, bn=_BN):
    return pl.BlockSpec((bn, width), lambda i: (i, 0))


def _edges(width):
    return pl.BlockSpec((_BE, width), lambda i: (i, 0))


def _f32(shape):
    return jax.ShapeDtypeStruct(shape, jnp.float32)


def kernel(local, pos, neighbours, resi, chain, batch, mask, params):
    p = params
    f32 = jnp.float32
    mf = mask.astype(f32)
    geo = jnp.concatenate(
        [pos.reshape(_N, 12), mf[:, None], jnp.zeros((_N, 3), f32),
         resi[:, None].astype(f32), chain[:, None].astype(f32),
         batch[:, None].astype(f32), jnp.zeros((_N, 5), f32)], axis=1)
    nb = neighbours.reshape(_NK).astype(jnp.int32)
    mf2 = mf[:, None]
    ch2 = chain[:, None].astype(jnp.int32)
    ba2 = batch[:, None].astype(jnp.int32)

    scat = np.zeros((24, 12), np.float32)
    for a in range(4):
        for c in range(3):
            scat[3 + c, a * 3 + c] = 1.0
    g3 = np.zeros((12, 4), np.float32)
    for a in range(4):
        for c in range(3):
            g3[a * 3 + c, a] = 1.0
    r16 = np.zeros((4, 64), np.float32)
    for a in range(4):
        r16[a, a * 16:(a + 1) * 16] = 1.0
    r3 = np.zeros((4, 12), np.float32)
    for a in range(4):
        r3[a, a * 3:(a + 1) * 3] = 1.0
    sm8 = np.zeros((24, 8), np.float32)
    sm8[12, :] = 1.0
    s_re = np.zeros((24, 1), np.float32); s_re[16, 0] = 1.0
    s_ch = np.zeros((24, 1), np.float32); s_ch[17, 0] = 1.0
    s_ba = np.zeros((24, 1), np.float32); s_ba[18, 0] = 1.0
    cc = np.tile(np.linspace(0.0, 22.0, 16, dtype=np.float32), 4)[None, :]
    i66 = np.arange(66, dtype=np.float32)[None, :]
    gh = np.zeros((128, _H), np.float32)
    for h in range(_H):
        gh[h * _DH:(h + 1) * _DH, h] = 1.0
    ght = gh.T.copy()

    def v(a):
        return a.reshape(1, -1)

    geo_e = _sc_gather(geo, nb, 24, f32)

    pair_ops = [geo_e, geo,
                p['W_relpos'], p['W_dist'], p['W_dir'],
                v(p['g_pln']), v(p['b_pln']),
                p['W_p1'], v(p['b_p1']), p['W_p2'], v(p['b_p2']), p['Wb'],
                scat, g3, r16, r3, sm8, s_re, s_ch, s_ba, cc, i66]
    pair_specs = [_edges(24), _nodes(24)] + [_full(a) for a in pair_ops[2:]]

    wqkv = jnp.concatenate([p['Wq'], p['Wk'], p['Wv']], axis=1)
    w4 = jnp.concatenate([p['W_upd'], p['W_lg'], p['W_cg'], p['W_bg']], axis=1)
    _BNQ = 400

    q, kv = pl.pallas_call(
        _qkv_body,
        grid=(_N // _BNQ,),
        in_specs=[_nodes(128, _BNQ), _full(v(p['g_ln1'])),
                  _full(v(p['b_ln1'])), _full(wqkv)],
        out_specs=[_nodes(128, _BNQ), _nodes(256, _BNQ)],
        out_shape=[_f32((_N, 128)),
                   jax.ShapeDtypeStruct((_N, 256), jnp.bfloat16)],
        compiler_params=_SEQ,
    )(local, v(p['g_ln1']), v(p['b_ln1']), wqkv)
    kvn = _sc_gather(kv, nb, 256, jnp.bfloat16)

    bias_e, pm_e = pl.pallas_call(
        _pair_body,
        grid=(_GRID,),
        in_specs=pair_specs,
        out_specs=[_edges(8), _edges(8)],
        out_shape=[_f32((_NK, 8)), _f32((_NK, 8))],
        compiler_params=_SEQ,
    )(*pair_ops)

    x = local
    for layer in range(_DEPTH):
        x, out4, sb, sc = pl.pallas_call(
            _attn_upd_body,
            grid=(_GRID,),
            in_specs=[_nodes(128), _edges(256), _edges(8), _edges(8),
                      _nodes(128), _nodes(1), _nodes(1), _nodes(1),
                      _full(p['Wo']), _full(gh), _full(ght),
                      _full(v(p['g_ln2'])), _full(v(p['b_ln2'])), _full(w4)],
            out_specs=[_nodes(128), _nodes(1024),
                       pl.BlockSpec((_NBA, 264), lambda i: (0, 0)),
                       pl.BlockSpec((_NCH, 264), lambda i: (0, 0))],
            out_shape=[_f32((_N, 128)), _f32((_N, 1024)),
                       _f32((_NBA, 264)), _f32((_NCH, 264))],
            compiler_params=_SEQ,
        )(q, kvn, bias_e, pm_e, x, mf2, ch2, ba2,
          p['Wo'], gh, ght, v(p['g_ln2']), v(p['b_ln2']), w4)

        if layer < _DEPTH - 1:
            x, q, kv = pl.pallas_call(
                _mix_qkv_body,
                grid=(_N // _BNQ,),
                in_specs=[_nodes(128, _BNQ), _nodes(1024, _BNQ),
                          _nodes(1, _BNQ), _nodes(1, _BNQ), _nodes(1, _BNQ),
                          _full(sb), _full(sc), _full(p['W_o2']),
                          _full(v(p['g_ln1'])), _full(v(p['b_ln1'])),
                          _full(wqkv)],
                out_specs=[_nodes(128, _BNQ), _nodes(128, _BNQ),
                           _nodes(256, _BNQ)],
                out_shape=[_f32((_N, 128)), _f32((_N, 128)),
                           jax.ShapeDtypeStruct((_N, 256), jnp.bfloat16)],
                compiler_params=_SEQ,
            )(x, out4, ch2, ba2, mf2, sb, sc, p['W_o2'],
              v(p['g_ln1']), v(p['b_ln1']), wqkv)
            kvn = _sc_gather(kv, nb, 256, jnp.bfloat16)
        else:
            x = pl.pallas_call(
                _mix_final_body,
                grid=(_GRID,),
                in_specs=[_nodes(128), _nodes(1024), _nodes(1), _nodes(1),
                          _nodes(1), _full(sb), _full(sc), _full(p['W_o2']),
                          _full(v(p['g_lnf'])), _full(v(p['b_lnf']))],
                out_specs=_nodes(128),
                out_shape=_f32((_N, 128)),
                compiler_params=_SEQ,
            )(x, out4, ch2, ba2, mf2, sb, sc, p['W_o2'],
              v(p['g_lnf']), v(p['b_lnf']))

    return x


# speedup vs baseline: 1.0098x; 1.0098x over previous
import functools

import jax
import jax.numpy as jnp
import numpy as np
from jax.experimental import pallas as pl
from jax.experimental.pallas import tpu as pltpu
from jax.experimental.pallas import tpu_sc as plsc

_N = 10000; _K = 32; _D = 128; _P = 64; _A = 4; _H = 8; _DH = 16
_NCH = 512; _NBA = 128; _DEPTH = 2
_NK = _N * _K
_BN = 200
_BE = _BN * _K
_GRID = _N // _BN
_CH = 80
_BNP = 400

_SEQ = pltpu.CompilerParams(dimension_semantics=("arbitrary",))


def _ln(x, g, b):
    m = jnp.mean(x, axis=-1, keepdims=True)
    v = jnp.mean((x - m) * (x - m), axis=-1, keepdims=True)
    return (x - m) * jax.lax.rsqrt(v + 1e-5) * g + b


def _expand(a, bn=_BN):
    return jnp.broadcast_to(a[:, None, :], (bn, _K, a.shape[-1])).reshape(bn * _K, a.shape[-1])


_WIN = 8


def _sc_gather(table, idx, width, dtype):
    nk = idx.shape[0]
    info = plsc.get_sparse_core_info()
    nc, ns = info.num_cores, info.num_subcores
    nw = nc * ns
    per = nk // nw
    steps = per // _CH
    mesh = plsc.VectorSubcoreMesh(core_axis_name="c", subcore_axis_name="s")

    @functools.partial(
        pl.kernel, mesh=mesh,
        compiler_params=pltpu.CompilerParams(use_tc_tiling_on_sc=False),
        out_type=jax.ShapeDtypeStruct((nk, width), dtype),
        scratch_types=[
            pltpu.VMEM((per,), jnp.int32),
            pltpu.VMEM((2, _CH, width), dtype),
            pltpu.SemaphoreType.DMA,
            pltpu.SemaphoreType.DMA,
            pltpu.SemaphoreType.DMA,
            pltpu.SemaphoreType.DMA,
        ],
    )
    def gather_k(tab_h, idx_h, out_h, idx_v, rows_v, g0, g1, w0, w1):
        wid = jax.lax.axis_index("s") * nc + jax.lax.axis_index("c")
        base = wid * per
        pltpu.sync_copy(idx_h.at[pl.ds(base, per)], idx_v)
        gsem = (g0, g1)
        wsem = (w0, w1)

        def gath(c, b):
            return pltpu.make_async_copy(
                tab_h.at[idx_v.at[pl.ds(c * _CH, _CH)]],
                rows_v.at[b], gsem[b])

        def wb(c, b):
            return pltpu.make_async_copy(
                rows_v.at[b], out_h.at[pl.ds(base + c * _CH, _CH)], wsem[b])

        def pair(g, carry):
            c0 = 2 * g
            for b in (0, 1):
                @pl.when(g > 0)
                def _(b=b):
                    wb(0, b).wait()
                gath(c0 + b, b).start()
            for b in (0, 1):
                gath(0, b).wait()
                wb(c0 + b, b).start()
            return carry

        jax.lax.fori_loop(0, steps // 2, pair, 0)
        if steps % 2:
            c = steps - 1
            wb(0, 0).wait()
            gath(c, 0).start()
            gath(0, 0).wait()
            wb(c, 0).start()
        for b in (0, 1):
            wb(0, b).wait()

    return gather_k(table, idx)


def _pair_body(geo_e, geo_d,
               wrel, wdist, wdir, gp, bp, wp1, bp1, wp2, bp2, wb,
               scat, g3, r16, r3, sm8, s_re, s_ch, s_ba, cc, i66,
               bias_o, pm_o):
    ge = geo_e[...]
    gd = geo_d[...]
    delta = ge - _expand(gd, _BNP)
    rel24 = jnp.clip(delta, -32.0, 32.0) + 32.0
    same24 = (delta == 0.0).astype(jnp.float32)
    relc = rel24 @ s_re[...]
    same = (same24 @ s_ch[...]) * (same24 @ s_ba[...])
    rel = jnp.where(same > 0.5, relc, 65.0)
    oh = (rel == i66[...]).astype(jnp.float32)
    pair = oh @ wrel[...]

    pos_e = ge[:, 0:12]
    ca12 = _expand(gd @ scat[...], _BNP)
    diff = pos_e - ca12
    d2 = (diff * diff) @ g3[...]
    dist = jnp.sqrt(d2)
    dist16 = dist @ r16[...]
    s = 22.0 / 16.0
    rbf = jnp.exp(-((dist16 - cc[...]) ** 2) * (1.0 / (2.0 * s * s)))
    pair = pair + rbf @ wdist[...]
    dirs = diff * ((1.0 / (dist + 1e-6)) @ r3[...])
    pair = pair + dirs @ wdir[...]

    pair = _ln(pair, gp[...], bp[...])
    pair = jax.nn.gelu(pair @ wp1[...] + bp1[...]) @ wp2[...] + bp2[...]
    bias_o[...] = pair @ wb[...]
    pm_o[...] = (ge @ sm8[...]) * _expand(gd @ sm8[...], _BNP)


def _qkv_body(x, gln1, bln1, wqkv, q_o, kv_o):
    h = _ln(x[...], gln1[...], bln1[...])
    qkv = h @ wqkv[...]
    q_o[...] = qkv[:, 0:128]
    kv_o[...] = qkv[:, 128:384].astype(jnp.bfloat16)


def _attn_upd_body(q, kvn, bias_e, pm_e, x, mf2, ch2, ba2,
                   wo, gh, ght, gln2, bln2, w4,
                   x_o, out4_o, sb_o, sc_o):
    qe = _expand(q[...])
    ke = kvn[:, 0:128].astype(jnp.float32)
    ve = kvn[:, 128:256].astype(jnp.float32)
    lg8 = ((qe * ke) @ gh[...]) * (1.0 / np.sqrt(_DH)) + bias_e[...]
    pm8 = pm_e[...]
    lg8 = jnp.where(pm8 > 0.0, lg8, -1e9)
    l3 = lg8.reshape(_BN, _K, _H)
    m3 = jnp.max(l3, axis=1, keepdims=True)
    e3 = jnp.exp(l3 - m3)
    s3 = jnp.sum(e3, axis=1, keepdims=True)
    att = (e3 / s3).reshape(_BE, _H) * pm8
    w = (att @ ght[...]) * ve
    o = w.reshape(_BN, _K, _D).sum(axis=1)
    mf = mf2[...]
    xn = x[...] + (o @ wo[...]) * mf
    x_o[...] = xn

    i = pl.program_id(0)
    h = _ln(xn, gln2[...], bln2[...])
    big = h @ w4[...]
    u = big[:, 0:256]
    gat = jax.nn.gelu(big[:, 256:1024])
    out4_o[...] = jnp.concatenate([u, gat], axis=1)
    u2 = jnp.concatenate([u * mf, jnp.broadcast_to(mf, (_BN, 8))], axis=1)
    ohb = (ba2[...] == jax.lax.broadcasted_iota(jnp.int32, (1, _NBA), 1)
           ).astype(jnp.float32)
    ohc = (ch2[...] == jax.lax.broadcasted_iota(jnp.int32, (1, _NCH), 1)
           ).astype(jnp.float32)
    pb = jax.lax.dot_general(ohb, u2, (((0,), (0,)), ((), ())), preferred_element_type=jnp.float32)
    pc = jax.lax.dot_general(ohc, u2, (((0,), (0,)), ((), ())), preferred_element_type=jnp.float32)

    @pl.when(i == 0)
    def _():
        sb_o[...] = jnp.zeros_like(sb_o)
        sc_o[...] = jnp.zeros_like(sc_o)

    sb_o[...] += pb
    sc_o[...] += pc


def _mix_core(x, out4, ch2, ba2, mf2, sb, sc, wo2):
    o4 = out4[...]
    u = o4[:, 0:256]
    lg = o4[:, 256:512]
    cg = o4[:, 512:768]
    bg = o4[:, 768:1024]
    sbv = sb[...]
    scv = sc[...]
    meanb = sbv[:, 0:256] / jnp.maximum(sbv[:, 256:257], 1.0)
    meanc = scv[:, 0:256] / jnp.maximum(scv[:, 256:257], 1.0)
    ohb = (ba2[...] == jax.lax.broadcasted_iota(jnp.int32, (1, _NBA), 1)
           ).astype(jnp.float32)
    ohc = (ch2[...] == jax.lax.broadcasted_iota(jnp.int32, (1, _NCH), 1)
           ).astype(jnp.float32)
    hidden = bg * (ohb @ meanb) + cg * (ohc @ meanc) + lg * u
    return x[...] + (hidden @ wo2[...]) * mf2[...]


def _mix_qkv_body(x, out4, ch2, ba2, mf2, sb, sc, wo2, gln1, bln1, wqkv,
                  x_o, q_o, kv_o):
    xo = _mix_core(x, out4, ch2, ba2, mf2, sb, sc, wo2)
    x_o[...] = xo
    h = _ln(xo, gln1[...], bln1[...])
    qkv = h @ wqkv[...]
    q_o[...] = qkv[:, 0:128]
    kv_o[...] = qkv[:, 128:384].astype(jnp.bfloat16)


def _mix_final_body(x, out4, ch2, ba2, mf2, sb, sc, wo2, glnf, blnf, x_o):
    xo = _mix_core(x, out4, ch2, ba2, mf2, sb, sc, wo2)
    x_o[...] = _ln(xo, glnf[...], blnf[...])


def _full(a):
    return pl.BlockSpec(a.shape, lambda i: (0,) * a.ndim)


def _nodes(width, bn=_BN):
    return pl.BlockSpec((bn, width), lambda i: (i, 0))


def _edges(width):
    return pl.BlockSpec((_BE, width), lambda i: (i, 0))


def _f32(shape):
    return jax.ShapeDtypeStruct(shape, jnp.float32)


def kernel(local, pos, neighbours, resi, chain, batch, mask, params):
    p = params
    f32 = jnp.float32
    mf = mask.astype(f32)
    geo = jnp.concatenate(
        [pos.reshape(_N, 12), mf[:, None], jnp.zeros((_N, 3), f32),
         resi[:, None].astype(f32), chain[:, None].astype(f32),
         batch[:, None].astype(f32), jnp.zeros((_N, 5), f32)], axis=1)
    nb = neighbours.reshape(_NK).astype(jnp.int32)
    mf2 = mf[:, None]
    ch2 = chain[:, None].astype(jnp.int32)
    ba2 = batch[:, None].astype(jnp.int32)

    scat = np.zeros((24, 12), np.float32)
    for a in range(4):
        for c in range(3):
            scat[3 + c, a * 3 + c] = 1.0
    g3 = np.zeros((12, 4), np.float32)
    for a in range(4):
        for c in range(3):
            g3[a * 3 + c, a] = 1.0
    r16 = np.zeros((4, 64), np.float32)
    for a in range(4):
        r16[a, a * 16:(a + 1) * 16] = 1.0
    r3 = np.zeros((4, 12), np.float32)
    for a in range(4):
        r3[a, a * 3:(a + 1) * 3] = 1.0
    sm8 = np.zeros((24, 8), np.float32)
    sm8[12, :] = 1.0
    s_re = np.zeros((24, 1), np.float32); s_re[16, 0] = 1.0
    s_ch = np.zeros((24, 1), np.float32); s_ch[17, 0] = 1.0
    s_ba = np.zeros((24, 1), np.float32); s_ba[18, 0] = 1.0
    cc = np.tile(np.linspace(0.0, 22.0, 16, dtype=np.float32), 4)[None, :]
    i66 = np.arange(66, dtype=np.float32)[None, :]
    gh = np.zeros((128, _H), np.float32)
    for h in range(_H):
        gh[h * _DH:(h + 1) * _DH, h] = 1.0
    ght = gh.T.copy()

    def v(a):
        return a.reshape(1, -1)

    geo_e = _sc_gather(geo, nb, 24, f32)

    pair_ops = [geo_e, geo,
                p['W_relpos'], p['W_dist'], p['W_dir'],
                v(p['g_pln']), v(p['b_pln']),
                p['W_p1'], v(p['b_p1']), p['W_p2'], v(p['b_p2']), p['Wb'],
                scat, g3, r16, r3, sm8, s_re, s_ch, s_ba, cc, i66]
    pair_specs = [pl.BlockSpec((_BNP * _K, 24), lambda i: (i, 0)),
                  _nodes(24, _BNP)] + [_full(a) for a in pair_ops[2:]]

    wqkv = jnp.concatenate([p['Wq'], p['Wk'], p['Wv']], axis=1)
    w4 = jnp.concatenate([p['W_upd'], p['W_lg'], p['W_cg'], p['W_bg']], axis=1)
    _BNQ = 400

    q, kv = pl.pallas_call(
        _qkv_body,
        grid=(_N // _BNQ,),
        in_specs=[_nodes(128, _BNQ), _full(v(p['g_ln1'])),
                  _full(v(p['b_ln1'])), _full(wqkv)],
        out_specs=[_nodes(128, _BNQ), _nodes(256, _BNQ)],
        out_shape=[_f32((_N, 128)),
                   jax.ShapeDtypeStruct((_N, 256), jnp.bfloat16)],
        compiler_params=_SEQ,
    )(local, v(p['g_ln1']), v(p['b_ln1']), wqkv)
    kvn = _sc_gather(kv, nb, 256, jnp.bfloat16)

    bias_e, pm_e = pl.pallas_call(
        _pair_body,
        grid=(_N // _BNP,),
        in_specs=pair_specs,
        out_specs=[pl.BlockSpec((_BNP * _K, 8), lambda i: (i, 0)),
                   pl.BlockSpec((_BNP * _K, 8), lambda i: (i, 0))],
        out_shape=[_f32((_NK, 8)), _f32((_NK, 8))],
        compiler_params=pltpu.CompilerParams(
            dimension_semantics=("arbitrary",),
            vmem_limit_bytes=100 * 1024 * 1024),
    )(*pair_ops)

    x = local
    for layer in range(_DEPTH):
        x, out4, sb, sc = pl.pallas_call(
            _attn_upd_body,
            grid=(_GRID,),
            in_specs=[_nodes(128), _edges(256), _edges(8), _edges(8),
                      _nodes(128), _nodes(1), _nodes(1), _nodes(1),
                      _full(p['Wo']), _full(gh), _full(ght),
                      _full(v(p['g_ln2'])), _full(v(p['b_ln2'])), _full(w4)],
            out_specs=[_nodes(128), _nodes(1024),
                       pl.BlockSpec((_NBA, 264), lambda i: (0, 0)),
                       pl.BlockSpec((_NCH, 264), lambda i: (0, 0))],
            out_shape=[_f32((_N, 128)), _f32((_N, 1024)),
                       _f32((_NBA, 264)), _f32((_NCH, 264))],
            compiler_params=_SEQ,
        )(q, kvn, bias_e, pm_e, x, mf2, ch2, ba2,
          p['Wo'], gh, ght, v(p['g_ln2']), v(p['b_ln2']), w4)

        if layer < _DEPTH - 1:
            x, q, kv = pl.pallas_call(
                _mix_qkv_body,
                grid=(_N // _BNQ,),
                in_specs=[_nodes(128, _BNQ), _nodes(1024, _BNQ),
                          _nodes(1, _BNQ), _nodes(1, _BNQ), _nodes(1, _BNQ),
                          _full(sb), _full(sc), _full(p['W_o2']),
                          _full(v(p['g_ln1'])), _full(v(p['b_ln1'])),
                          _full(wqkv)],
                out_specs=[_nodes(128, _BNQ), _nodes(128, _BNQ),
                           _nodes(256, _BNQ)],
                out_shape=[_f32((_N, 128)), _f32((_N, 128)),
                           jax.ShapeDtypeStruct((_N, 256), jnp.bfloat16)],
                compiler_params=_SEQ,
            )(x, out4, ch2, ba2, mf2, sb, sc, p['W_o2'],
              v(p['g_ln1']), v(p['b_ln1']), wqkv)
            kvn = _sc_gather(kv, nb, 256, jnp.bfloat16)
        else:
            x = pl.pallas_call(
                _mix_final_body,
                grid=(_GRID,),
                in_specs=[_nodes(128), _nodes(1024), _nodes(1), _nodes(1),
                          _nodes(1), _full(sb), _full(sc), _full(p['W_o2']),
                          _full(v(p['g_lnf'])), _full(v(p['b_lnf']))],
                out_specs=_nodes(128),
                out_shape=_f32((_N, 128)),
                compiler_params=_SEQ,
            )(x, out4, ch2, ba2, mf2, sb, sc, p['W_o2'],
              v(p['g_lnf']), v(p['b_lnf']))

    return x
